# baseline (speedup 1.0000x reference)
import functools
import jax
import jax.numpy as jnp
from jax import lax
from jax.experimental import pallas as pl
from jax.experimental.pallas import tpu as pltpu
from jax.experimental.pallas import tpu_sc as plsc

VOCAB = 1000000
DIM = 64
BATCH = 16384

SC_ROWS = 8192
TC_ROWS = BATCH - SC_ROWS

NC = 2
NS = 16
L = 16
NW = NC * NS
BPW = SC_ROWS // NW
NSEM = 4

_mesh = plsc.VectorSubcoreMesh(
    core_axis_name="c", subcore_axis_name="s", num_cores=NC, num_subcores=NS
)


@functools.partial(
    pl.kernel,
    mesh=_mesh,
    out_type=jax.ShapeDtypeStruct((SC_ROWS, DIM), jnp.float32),
    scratch_types=[
        pltpu.VMEM((BPW,), jnp.int32),
        pltpu.VMEM((BPW,), jnp.int32),
        pltpu.VMEM((BPW, DIM), jnp.float32),
        pltpu.VMEM((BPW, DIM), jnp.float32),
        [pltpu.SemaphoreType.DMA] * NSEM,
        [pltpu.SemaphoreType.DMA] * NSEM,
        pltpu.SemaphoreType.DMA,
    ],
    cost_estimate=pl.CostEstimate(
        flops=1 << 20, bytes_accessed=1 << 30, transcendentals=0),
)
def _sc_sign_dot(mw_hbm, cw_hbm, mt_hbm, ct_hbm, out_hbm,
                 mw_v, cw_v, mrows_v, crows_v, msems, csems, osem):
    wid = lax.axis_index("s") * NC + lax.axis_index("c")
    base = wid * BPW

    pltpu.sync_copy(mw_hbm.at[pl.ds(base, BPW)], mw_v)
    pltpu.sync_copy(cw_hbm.at[pl.ds(base, BPW)], cw_v)

    def issue(g, carry):
        r0 = g * L
        mv = mw_v[pl.ds(r0, L)]
        cv = cw_v[pl.ds(r0, L)]
        for lane in range(L):
            pltpu.async_copy(
                mt_hbm.at[pl.ds(mv[lane], 1)],
                mrows_v.at[pl.ds(r0 + lane, 1)], msems[lane % NSEM])
            pltpu.async_copy(
                ct_hbm.at[pl.ds(cv[lane], 1)],
                crows_v.at[pl.ds(r0 + lane, 1)], csems[lane % NSEM])
        return carry
    lax.fori_loop(0, BPW // L, issue, 0)

    for s in range(NSEM):
        pltpu.make_async_copy(
            mt_hbm.at[pl.ds(0, BPW // NSEM)],
            mrows_v.at[pl.ds(0, BPW // NSEM)], msems[s]).wait()
        pltpu.make_async_copy(
            ct_hbm.at[pl.ds(0, BPW // NSEM)],
            crows_v.at[pl.ds(0, BPW // NSEM)], csems[s]).wait()

    def body(r, carry):
        for c in range(DIM // L):
            a = mrows_v[r, pl.ds(c * L, L)]
            b = crows_v[r, pl.ds(c * L, L)]
            prod = a * b
            mrows_v[r, pl.ds(c * L, L)] = prod / jnp.abs(prod)
        return carry
    lax.fori_loop(0, BPW, body, 0)

    pltpu.async_copy(mrows_v, out_hbm.at[pl.ds(base, BPW)], osem).wait()


BR = 256
NBLK = TC_ROWS // BR


def _tc_body(mw_sm, cw_sm, mt_hbm, ct_hbm, out_vmem, mbuf, cbuf, msem, csem):
    i = pl.program_id(0)
    base = i * BR

    def issue(r, carry):
        v = mw_sm[base + r]
        pltpu.make_async_copy(
            mt_hbm.at[pl.ds(v, 1)], mbuf.at[pl.ds(r, 1)], msem).start()
        w = cw_sm[base + r]
        pltpu.make_async_copy(
            ct_hbm.at[pl.ds(w, 1)], cbuf.at[pl.ds(r, 1)], csem).start()
        return carry
    lax.fori_loop(0, BR, issue, 0, unroll=8)

    pltpu.make_async_copy(mt_hbm.at[pl.ds(0, BR)], mbuf, msem).wait()
    pltpu.make_async_copy(ct_hbm.at[pl.ds(0, BR)], cbuf, csem).wait()

    prod = mbuf[...] * cbuf[...]
    out_vmem[...] = prod / jnp.abs(prod)


def _tc_gather_sign(mw, cw, mt, ct):
    grid_spec = pltpu.PrefetchScalarGridSpec(
        num_scalar_prefetch=2,
        grid=(NBLK,),
        in_specs=[
            pl.BlockSpec(memory_space=pltpu.HBM),
            pl.BlockSpec(memory_space=pltpu.HBM),
        ],
        out_specs=pl.BlockSpec((BR, DIM), lambda i, *_: (i, 0)),
        scratch_shapes=[
            pltpu.VMEM((BR, DIM), jnp.float32),
            pltpu.VMEM((BR, DIM), jnp.float32),
            pltpu.SemaphoreType.DMA,
            pltpu.SemaphoreType.DMA,
        ],
    )
    return pl.pallas_call(
        _tc_body,
        grid_spec=grid_spec,
        out_shape=jax.ShapeDtypeStruct((TC_ROWS, DIM), jnp.float32),
    )(mw, cw, mt, ct)


def kernel(main_words, ctx_words, main_table, ctx_table):
    mw = main_words.astype(jnp.int32)
    cw = ctx_words.astype(jnp.int32)
    out_sc = _sc_sign_dot(mw[:SC_ROWS], cw[:SC_ROWS], main_table, ctx_table)
    out_tc = _tc_gather_sign(mw[SC_ROWS:], cw[SC_ROWS:], main_table, ctx_table)
    return jnp.concatenate([out_sc, out_tc], axis=0)


# speedup vs baseline: 1.1190x; 1.1190x over previous
import functools
import jax
import jax.numpy as jnp
from jax import lax
from jax.experimental import pallas as pl
from jax.experimental.pallas import tpu as pltpu
from jax.experimental.pallas import tpu_sc as plsc

VOCAB = 1000000
DIM = 64
BATCH = 16384

NC = 2
NS = 16
L = 16
NW = NC * NS
BPW = BATCH // NW
CR = 128
NPASS = BPW // CR

_mesh = plsc.VectorSubcoreMesh(
    core_axis_name="c", subcore_axis_name="s", num_cores=NC, num_subcores=NS
)


@functools.partial(
    pl.kernel,
    mesh=_mesh,
    out_type=jax.ShapeDtypeStruct((BATCH, DIM), jnp.float32),
    scratch_types=[
        pltpu.VMEM((BPW,), jnp.int32),
        pltpu.VMEM((BPW,), jnp.int32),
        [pltpu.VMEM((CR, DIM), jnp.float32)] * 2,
        [pltpu.VMEM((CR, DIM), jnp.float32)] * 2,
        [pltpu.SemaphoreType.DMA] * 2,
        [pltpu.SemaphoreType.DMA] * 2,
        pltpu.SemaphoreType.DMA,
    ],
)
def _sc_sign_dot(mw_hbm, cw_hbm, mt_hbm, ct_hbm, out_hbm,
                 mw_v, cw_v, mrows, crows, msems, csems, osem):
    wid = lax.axis_index("s") * NC + lax.axis_index("c")
    base = wid * BPW

    pltpu.sync_copy(mw_hbm.at[pl.ds(base, BPW)], mw_v)
    pltpu.sync_copy(cw_hbm.at[pl.ds(base, BPW)], cw_v)

    def issue_pass(p, slot):
        def issue(g, carry):
            r0 = g * L
            mv = mw_v[pl.ds(p * CR + r0, L)]
            cv = cw_v[pl.ds(p * CR + r0, L)]
            for lane in range(L):
                pltpu.async_copy(
                    mt_hbm.at[pl.ds(mv[lane], 1)],
                    mrows[slot].at[pl.ds(r0 + lane, 1)], msems[slot])
                pltpu.async_copy(
                    ct_hbm.at[pl.ds(cv[lane], 1)],
                    crows[slot].at[pl.ds(r0 + lane, 1)], csems[slot])
            return carry
        lax.fori_loop(0, CR // L, issue, 0)

    issue_pass(0, 0)
    for p in range(NPASS):
        slot = p % 2
        pltpu.make_async_copy(
            mt_hbm.at[pl.ds(0, CR)], mrows[slot], msems[slot]).wait()
        pltpu.make_async_copy(
            ct_hbm.at[pl.ds(0, CR)], crows[slot], csems[slot]).wait()

        if p + 1 < NPASS:
            issue_pass(p + 1, 1 - slot)

        def body(r, carry):
            for c in range(DIM // L):
                a = mrows[slot][r, pl.ds(c * L, L)]
                b = crows[slot][r, pl.ds(c * L, L)]
                prod = a * b
                mrows[slot][r, pl.ds(c * L, L)] = prod / jnp.abs(prod)
            return carry
        lax.fori_loop(0, CR, body, 0)

        pltpu.async_copy(
            mrows[slot], out_hbm.at[pl.ds(base + p * CR, CR)], osem).wait()


def kernel(main_words, ctx_words, main_table, ctx_table):
    return _sc_sign_dot(main_words.astype(jnp.int32), ctx_words.astype(jnp.int32),
                        main_table, ctx_table)
